# baseline (speedup 1.0000x reference)
import functools

import jax
import jax.numpy as jnp
from jax import lax
from jax.experimental import pallas as pl
from jax.experimental.pallas import tpu as pltpu

MARGIN_ = 0.02
BT_ = 128


def _loss_kernel(n, nb, u_col, r_col, u_row, r_row, loss_ref, numacc):
    ib = pl.program_id(0)

    @pl.when(ib == 0)
    def _init():
        numacc[:, :] = jnp.zeros_like(numacc)

    rc = r_col[pl.ds(ib * BT_, BT_), :]
    uc = u_col[pl.ds(ib * BT_, BT_), :]
    m128 = jnp.broadcast_to(MARGIN_ - rc, (BT_, BT_))
    u128 = jnp.broadcast_to(uc, (BT_, BT_))

    rr_d = r_row[:, pl.ds(ib * BT_, BT_)]
    ur_d = u_row[:, pl.ds(ib * BT_, BT_)]
    e_d = jnp.maximum(m128 + rr_d, 0.0)
    ri = lax.broadcasted_iota(jnp.int32, (BT_, BT_), 0)
    ci = lax.broadcasted_iota(jnp.int32, (BT_, BT_), 1)
    w_d = jnp.where(ci > ri, u128 - ur_d, 0.0)
    acc0 = e_d * w_d

    def body(jb, acc):
        rr = r_row[:, pl.ds(jb * BT_, BT_)]
        ur = u_row[:, pl.ds(jb * BT_, BT_)]
        e = jnp.maximum(m128 + rr, 0.0)
        w = u128 - ur
        return acc + e * w

    acc = lax.fori_loop(ib + 1, nb, body, acc0)
    numacc[:, :] += acc

    @pl.when(ib == nb - 1)
    def _final():
        num = jnp.sum(numacc[:, :])
        idx = lax.broadcasted_iota(jnp.int32, (1, n), 1)
        coef = ((n - 1) - 2 * idx).astype(jnp.float32)
        den = jnp.sum(u_row[:, :] * coef)
        loss_ref[0, 0] = num / (den + 1e-8)


@jax.jit
def kernel(pred, y_true):
    p = pred.reshape(-1).astype(jnp.float32)
    y = y_true.reshape(-1).astype(jnp.float32)
    n = p.shape[0]
    nb = n // BT_

    neg_u, r = lax.sort((-y, p), num_keys=1)
    u = -neg_u

    loss = pl.pallas_call(
        functools.partial(_loss_kernel, n, nb),
        grid=(nb,),
        in_specs=[
            pl.BlockSpec((n, 1), lambda i: (0, 0)),
            pl.BlockSpec((n, 1), lambda i: (0, 0)),
            pl.BlockSpec((1, n), lambda i: (0, 0)),
            pl.BlockSpec((1, n), lambda i: (0, 0)),
        ],
        out_specs=pl.BlockSpec(memory_space=pltpu.SMEM),
        out_shape=jax.ShapeDtypeStruct((1, 1), jnp.float32),
        scratch_shapes=[
            pltpu.VMEM((BT_, BT_), jnp.float32),
        ],
    )(u.reshape(n, 1), r.reshape(n, 1), u.reshape(1, n), r.reshape(1, n))

    return loss[0, 0]
